# baseline (speedup 1.0000x reference)
import functools

import jax
import jax.numpy as jnp
from jax import lax
from jax.experimental import pallas as pl
from jax.experimental.pallas import tpu as pltpu
from jax.experimental.pallas import tpu_sc as plsc

_VOCAB = 1_000_000
_D = 32
_L = 24
_B = 16384
_C = 20

_NC = 2
_NS = 16
_NW = _NC * _NS
_BW = _B // _NW
_FILL = 40.0
_IDMASK = (1 << 20) - 1
_FLMASK = (1 << 21) - 1
_LENCOL = 24
_CS = 16
_NCHUNK = _BW // _CS


def _sc_body(ctxi_hbm, tgt_hbm, inemb_hbm, nodemb_hbm, pc_hbm, out_hbm,
             tgt_v, pcidx, pc_big, lens_v, flat, ctxw_v, dmaidx, big_f,
             mean_t_v, sc_out, sem):
  wid = lax.axis_index("s") * _NC + lax.axis_index("c")
  base = wid * _BW
  iota = lax.iota(jnp.int32, 16)
  zeros = jnp.zeros((16,), jnp.float32)
  inv_c = jnp.float32(1.0 / _C)

  pltpu.sync_copy(tgt_hbm.at[pl.ds(base, _BW)], tgt_v)

  def tgt_idx(i, carry):
    pcidx[pl.ds(i * 16, 16)] = lax.shift_right_logical(
        tgt_v[pl.ds(i * 16, 16)], 2)
    return carry

  lax.fori_loop(0, _BW // 16, tgt_idx, 0)

  def pc_chunk(j, carry):
    pltpu.async_copy(pc_hbm.at[pcidx.at[pl.ds(j * 128, 128)]], pc_big,
                     sem).wait()

    def pblock(b, c2):
      off = j * 128 + b * 16
      lanes_loc = b * 16 + iota
      colbase = (tgt_v[pl.ds(off, 16)] & 3) * 32
      lens_v[pl.ds(off, 16)] = plsc.load_gather(
          pc_big, [lanes_loc, colbase + _LENCOL])
      glanes = off + iota

      def pl_body(l, c3):
        pv = plsc.load_gather(pc_big, [lanes_loc, colbase + l])
        pos = glanes * _L + l
        plsc.store_scatter(flat, [pos >> 7, pos & 127], pv & _FLMASK)
        return c3

      lax.fori_loop(0, _L, pl_body, 0)
      return c2

    lax.fori_loop(0, 8, pblock, 0)
    return carry

  lax.fori_loop(0, _BW // 128, pc_chunk, 0)

  def ctx_chunk(k, carry):
    pltpu.sync_copy(ctxi_hbm.at[pl.ds(base * _C + k * _CS * _C, _CS * _C)],
                    ctxw_v)

    def cidx(i, c2):
      dmaidx[pl.ds(i * 16, 16)] = lax.shift_right_logical(
          ctxw_v[pl.ds(i * 16, 16)], 2)
      return c2

    lax.fori_loop(0, _CS * _C // 16, cidx, 0)
    d1 = pltpu.async_copy(inemb_hbm.at[dmaidx.at[pl.ds(0, 128)]],
                          big_f.at[pl.ds(0, 128)], sem)
    d2 = pltpu.async_copy(inemb_hbm.at[dmaidx.at[pl.ds(128, 128)]],
                          big_f.at[pl.ds(128, 128)], sem)
    d3 = pltpu.async_copy(inemb_hbm.at[dmaidx.at[pl.ds(256, 64)]],
                          big_f.at[pl.ds(256, 64)], sem)
    d1.wait()
    d2.wait()
    d3.wait()

    def acc_c(c, accs):
      wv = plsc.load_gather(ctxw_v, [iota * _C + c])
      colb = (wv & 3) * 32
      rowv = iota * _C + c
      return tuple(accs[d] + plsc.load_gather(big_f, [rowv, colb + d])
                   for d in range(_D))

    accs = lax.fori_loop(0, _C, acc_c, (zeros,) * _D)
    for d in range(_D):
      plsc.store_scatter(mean_t_v, [jnp.full((16,), d, jnp.int32),
                                    k * _CS + iota], accs[d] * inv_c)
    return carry

  lax.fori_loop(0, _NCHUNK, ctx_chunk, 0)

  def node_chunk(k, carry):
    def nidx(r, c2):
      for j2 in range(8):
        fl = flat[3 * k + r, pl.ds(j2 * 16, 16)]
        dmaidx[pl.ds(r * 128 + j2 * 16, 16)] = lax.shift_right_logical(
            fl & _IDMASK, 2)
      return c2

    lax.fori_loop(0, 3, nidx, 0)
    d1 = pltpu.async_copy(nodemb_hbm.at[dmaidx.at[pl.ds(0, 128)]],
                          big_f.at[pl.ds(0, 128)], sem)
    d2 = pltpu.async_copy(nodemb_hbm.at[dmaidx.at[pl.ds(128, 128)]],
                          big_f.at[pl.ds(128, 128)], sem)
    d3 = pltpu.async_copy(nodemb_hbm.at[dmaidx.at[pl.ds(256, 128)]],
                          big_f.at[pl.ds(256, 128)], sem)
    d1.wait()
    d2.wait()
    d3.wait()

    lanes = k * _CS + iota
    lens_t = plsc.load_gather(lens_v, [lanes])
    mean_t = [
        plsc.load_gather(mean_t_v, [jnp.full((16,), d, jnp.int32), lanes])
        for d in range(_D)
    ]

    def l_body(l, c2, lens_t=lens_t, mean_t=mean_t, lanes=lanes):
      pos = lanes * _L + l
      fl = plsc.load_gather(flat, [pos >> 7, pos & 127])
      colv = (fl & 3) * 32
      code = lax.shift_right_logical(fl, 20)
      rowv = iota * _L + l
      acc = zeros
      for d in range(_D):
        acc = acc + mean_t[d] * plsc.load_gather(big_f, [rowv, colv + d])
      sign = code.astype(jnp.float32) * 2.0 - 1.0
      val = jnp.where(jnp.full((16,), l, jnp.int32) < lens_t, sign * acc,
                      _FILL)
      plsc.store_scatter(sc_out, [iota, jnp.full((16,), l, jnp.int32)], val)
      return c2

    lax.fori_loop(0, _L, l_body, 0)
    pltpu.sync_copy(sc_out, out_hbm.at[pl.ds(base + k * _CS, _CS)])
    return carry

  lax.fori_loop(0, _NCHUNK, node_chunk, 0)


_sc_scores = functools.partial(
    pl.kernel,
    out_type=jax.ShapeDtypeStruct((_B, _L), jnp.float32),
    mesh=plsc.VectorSubcoreMesh(core_axis_name="c", subcore_axis_name="s"),
    compiler_params=pltpu.CompilerParams(use_tc_tiling_on_sc=True,
                                         needs_layout_passes=False),
    scratch_types=[
        pltpu.VMEM((_BW,), jnp.int32),
        pltpu.VMEM((_BW,), jnp.int32),
        pltpu.VMEM((128, 128), jnp.int32),
        pltpu.VMEM((_BW,), jnp.int32),
        pltpu.VMEM((_BW * _L // 128, 128), jnp.int32),
        pltpu.VMEM((_CS * _C,), jnp.int32),
        pltpu.VMEM((_CS * _L,), jnp.int32),
        pltpu.VMEM((_CS * _L, 128), jnp.float32),
        pltpu.VMEM((_D, _BW), jnp.float32),
        pltpu.VMEM((_CS, _L), jnp.float32),
        pltpu.SemaphoreType.DMA,
    ],
)(_sc_body)


def _loss_body(x_ref, o_ref):
  x = x_ref[...]
  loss = jnp.log(1.0 + jnp.exp(-jnp.abs(x))) - jnp.minimum(x, 0.0)
  o_ref[0, 0] = jnp.sum(loss) * jnp.float32(1.0 / _B)


_loss = pl.pallas_call(
    _loss_body,
    out_shape=jax.ShapeDtypeStruct((1, 1), jnp.float32),
    out_specs=pl.BlockSpec(memory_space=pltpu.SMEM),
)


@jax.jit
def _impl(context_words, target_words, input_emb, internal_emb, paths, codes,
          path_lens):
  ctx1d = context_words.astype(jnp.int32).reshape(-1)
  tgt1d = target_words.astype(jnp.int32)
  probe = jax.lax.optimization_barrier(
      input_emb.reshape(_VOCAB // 32, 4, 8, _D).transpose(0, 2, 1, 3)
      .reshape(_VOCAB // 4, 4 * _D))
  probe = probe.reshape(_VOCAB // 32, 8, 4, _D).transpose(0, 2, 1, 3)
  inemb_b = probe.reshape(_VOCAB, _D).reshape(_VOCAB // 4, 4 * _D)
  nodemb_b = jnp.concatenate(
      [internal_emb, jnp.zeros((1, _D), jnp.float32)],
      axis=0).reshape(_VOCAB // 4, 4 * _D)
  pc = paths.astype(jnp.int32) | (codes.astype(jnp.int32) << 20)
  packed = jnp.concatenate(
      [pc, path_lens.astype(jnp.int32)[:, None],
       jnp.zeros((_VOCAB, _D - _L - 1), jnp.int32)],
      axis=1).reshape(_VOCAB // 4, 4 * _D)
  scores = _sc_scores(ctx1d, tgt1d, inemb_b, nodemb_b, packed)
  loss = _loss(scores.reshape(_B * _L // 128, 128))
  return loss[0, 0]


def kernel(context_words, target_words, input_emb, internal_emb, paths, codes,
           path_lens):
  return _impl(context_words, target_words, input_emb, internal_emb, paths,
               codes, path_lens)


# speedup vs baseline: 1.1831x; 1.1831x over previous
import functools

import jax
import jax.numpy as jnp
from jax import lax
from jax.experimental import pallas as pl
from jax.experimental.pallas import tpu as pltpu
from jax.experimental.pallas import tpu_sc as plsc

_VOCAB = 1_000_000
_D = 32
_L = 24
_B = 16384
_C = 20

_NC = 2
_NS = 16
_NW = _NC * _NS
_BW = _B // _NW
_SUB = 32
_NSUB = _BW // _SUB
_CHUNK = 128
_FILL = 40.0
_IDMASK = (1 << 20) - 1
_LENCOL = 24


def _sc_body(ctxi_hbm, tgt_hbm, inemb_hbm, nodemb_hbm, pc_hbm, out_hbm,
             tgt_v, pc_v, flat_idx, ctx_idx, mean_v, ctx_rows, node_rows,
             scores_v, sem):
  wid = lax.axis_index("s") * _NC + lax.axis_index("c")
  base = wid * _BW
  iota = lax.iota(jnp.int32, 16)
  zeros = jnp.zeros((16,), jnp.float32)

  pltpu.sync_copy(tgt_hbm.at[pl.ds(wid * (_BW // _CHUNK), _BW // _CHUNK)],
                  tgt_v)
  pltpu.sync_copy(
      ctxi_hbm.at[pl.ds(wid * (_BW * _C // _CHUNK), _BW * _C // _CHUNK)],
      ctx_idx)

  descs = []
  for j in range(_BW // _CHUNK):
    descs.append(pltpu.async_copy(
        pc_hbm.at[tgt_v.at[j]], pc_v.at[pl.ds(j * _CHUNK, _CHUNK)], sem))
  for d in descs:
    d.wait()

  def flat_body(r8, carry):
    for j in range(8):
      f = r8 * _CHUNK + j * 16 + iota
      v = plsc.load_gather(pc_v, [f // _L, f % _L])
      flat_idx[r8, pl.ds(j * 16, 16)] = v & _IDMASK
    return carry

  lax.fori_loop(0, _BW * _L // _CHUNK, flat_body, 0)

  inv_c = jnp.float32(1.0 / _C)

  def ctx_chunk(sc, carry):
    cdescs = []
    for j in range(_SUB * _C // _CHUNK):
      r0 = sc * (_SUB * _C // _CHUNK) + j
      cdescs.append(pltpu.async_copy(
          inemb_hbm.at[ctx_idx.at[r0]],
          ctx_rows.at[pl.ds(j * _CHUNK, _CHUNK)], sem))
    for d in cdescs:
      d.wait()

    def sample_body(s, c2):
      rbase = s * _C
      acc_e = zeros
      acc_o = zeros
      for c in range(_C):
        w = ctx_rows[rbase + c, pl.ds(0, 16)]
        acc_e = acc_e + plsc.bitcast(lax.shift_left(w, 16), jnp.float32)
        acc_o = acc_o + plsc.bitcast(w & jnp.int32(-65536), jnp.float32)
      g = sc * _SUB + s
      mean_v[g, pl.ds(0, 16)] = acc_e * inv_c
      mean_v[g, pl.ds(16, 16)] = acc_o * inv_c
      return c2

    lax.fori_loop(0, _SUB, sample_body, 0)
    return carry

  lax.fori_loop(0, _NSUB, ctx_chunk, 0)

  def node_chunk(sc, carry):
    ndescs = []
    for j in range(_SUB * _L // _CHUNK):
      r0 = sc * (_SUB * _L // _CHUNK) + j
      ndescs.append(pltpu.async_copy(
          nodemb_hbm.at[flat_idx.at[r0]],
          node_rows.at[pl.ds(j * _CHUNK, _CHUNK)], sem))
    for d in ndescs:
      d.wait()

    for blk in range(_SUB // 16):
      s0 = sc * _SUB + blk * 16
      lanes = s0 + iota
      lens_t = plsc.load_gather(pc_v, [lanes, jnp.full((16,), _LENCOL,
                                                       jnp.int32)])
      mean_t = [
          plsc.load_gather(mean_v, [lanes, jnp.full((16,), d_, jnp.int32)])
          for d_ in range(_D)
      ]
      row0 = (blk * 16 + iota) * _L

      def l_body(l, c2, row0=row0, lanes=lanes, lens_t=lens_t, mean_t=mean_t):
        lv = jnp.full((16,), l, jnp.int32)
        acc = zeros
        for p in range(_D // 2):
          wv = plsc.load_gather(node_rows,
                                [row0 + l, jnp.full((16,), p, jnp.int32)])
          ne = plsc.bitcast(lax.shift_left(wv, 16), jnp.float32)
          no = plsc.bitcast(wv & jnp.int32(-65536), jnp.float32)
          acc = acc + mean_t[p] * ne + mean_t[16 + p] * no
        code = lax.shift_right_logical(plsc.load_gather(pc_v, [lanes, lv]),
                                       20) & 1
        sign = code.astype(jnp.float32) * 2.0 - 1.0
        val = jnp.where(lv < lens_t, sign * acc, _FILL)
        plsc.store_scatter(scores_v, [lanes, lv], val)
        return c2

      lax.fori_loop(0, _L, l_body, 0)
    return carry

  lax.fori_loop(0, _NSUB, node_chunk, 0)

  pltpu.sync_copy(scores_v, out_hbm.at[pl.ds(base, _BW)])


_sc_scores = functools.partial(
    pl.kernel,
    out_type=jax.ShapeDtypeStruct((_B, _L), jnp.float32),
    mesh=plsc.VectorSubcoreMesh(core_axis_name="c", subcore_axis_name="s"),
    compiler_params=pltpu.CompilerParams(use_tc_tiling_on_sc=False,
                                         needs_layout_passes=False),
    scratch_types=[
        pltpu.VMEM((_BW // _CHUNK, _CHUNK), jnp.int32),
        pltpu.VMEM((_BW, _D), jnp.int32),
        pltpu.VMEM((_BW * _L // _CHUNK, _CHUNK), jnp.int32),
        pltpu.VMEM((_BW * _C // _CHUNK, _CHUNK), jnp.int32),
        pltpu.VMEM((_BW, _D), jnp.float32),
        pltpu.VMEM((_SUB * _C, _D // 2), jnp.int32),
        pltpu.VMEM((_SUB * _L, _D // 2), jnp.int32),
        pltpu.VMEM((_BW, _L), jnp.float32),
        pltpu.SemaphoreType.DMA,
    ],
)(_sc_body)


def _loss_body(x_ref, o_ref):
  x = x_ref[...]
  loss = jnp.log(1.0 + jnp.exp(-jnp.abs(x))) - jnp.minimum(x, 0.0)
  o_ref[0, 0] = jnp.sum(loss) * jnp.float32(1.0 / _B)


_loss = pl.pallas_call(
    _loss_body,
    out_shape=jax.ShapeDtypeStruct((1, 1), jnp.float32),
    out_specs=pl.BlockSpec(memory_space=pltpu.SMEM),
)


@jax.jit
def _impl(context_words, target_words, input_emb, internal_emb, paths, codes,
          path_lens):
  ctx_flat = context_words.astype(jnp.int32).reshape(_B * _C // _CHUNK, _CHUNK)
  tgt = target_words.astype(jnp.int32).reshape(_B // _CHUNK, _CHUNK)
  inemb_p = lax.bitcast_convert_type(
      input_emb.astype(jnp.bfloat16).reshape(_VOCAB, _D // 2, 2), jnp.int32)
  nodemb_p = lax.bitcast_convert_type(
      internal_emb.astype(jnp.bfloat16).reshape(_VOCAB - 1, _D // 2, 2),
      jnp.int32)
  pc = paths.astype(jnp.int32) | (codes.astype(jnp.int32) << 20)
  packed = jnp.concatenate(
      [pc, path_lens.astype(jnp.int32)[:, None],
       jnp.zeros((_VOCAB, _D - _L - 1), jnp.int32)], axis=1)
  scores = _sc_scores(ctx_flat, tgt, inemb_p, nodemb_p, packed)
  loss = _loss(scores.reshape(_B * _L // _CHUNK, _CHUNK))
  return loss[0, 0]


def kernel(context_words, target_words, input_emb, internal_emb, paths, codes,
           path_lens):
  return _impl(context_words, target_words, input_emb, internal_emb, paths,
               codes, path_lens)


# speedup vs baseline: 1.7620x; 1.4893x over previous
import functools

import jax
import jax.numpy as jnp
from jax import lax
from jax.experimental import pallas as pl
from jax.experimental.pallas import tpu as pltpu
from jax.experimental.pallas import tpu_sc as plsc

_VOCAB = 1_000_000
_D = 32
_L = 24
_B = 16384
_C = 20

_NC = 2
_NS = 16
_NW = _NC * _NS
_BW = _B // _NW
_SUB = 32
_NSUB = _BW // _SUB
_CHUNK = 128
_FILL = 40.0
_IDMASK = (1 << 20) - 1
_LENCOL = 24


def _sc_body(ctxi_hbm, tgt_hbm, inemb_hbm, nodemb_hbm, paths_hbm, cbits_hbm,
             lens_hbm, out_hbm, tgt_v, paths_v, cb_v, lens_v, flat_idx,
             ctx_idx, mean_v, ctx_rows, node_rows, scores_v, sem):
  wid = lax.axis_index("s") * _NC + lax.axis_index("c")
  base = wid * _BW
  iota = lax.iota(jnp.int32, 16)
  zeros = jnp.zeros((16,), jnp.float32)

  pltpu.sync_copy(tgt_hbm.at[pl.ds(wid * (_BW // _CHUNK), _BW // _CHUNK)],
                  tgt_v)
  pltpu.sync_copy(
      ctxi_hbm.at[pl.ds(wid * (_BW * _C // _CHUNK), _BW * _C // _CHUNK)],
      ctx_idx)

  descs = []
  for j in range(_BW // _CHUNK):
    idx = tgt_v.at[j]
    descs.append(pltpu.async_copy(
        paths_hbm.at[idx], paths_v.at[pl.ds(j * _CHUNK, _CHUNK)], sem))
    descs.append(pltpu.async_copy(cbits_hbm.at[idx], cb_v.at[j], sem))
    descs.append(pltpu.async_copy(lens_hbm.at[idx], lens_v.at[j], sem))
  for d in descs:
    d.wait()

  def flat_body(r8, carry):
    for j in range(8):
      f = r8 * _CHUNK + j * 16 + iota
      v = plsc.load_gather(paths_v, [f // _L, f % _L])
      flat_idx[r8, pl.ds(j * 16, 16)] = v
    return carry

  lax.fori_loop(0, _BW * _L // _CHUNK, flat_body, 0)

  inv_c = jnp.float32(1.0 / _C)

  def ctx_chunk(sc, carry):
    cdescs = []
    for j in range(_SUB * _C // _CHUNK):
      r0 = sc * (_SUB * _C // _CHUNK) + j
      cdescs.append(pltpu.async_copy(
          inemb_hbm.at[ctx_idx.at[r0]],
          ctx_rows.at[pl.ds(j * _CHUNK, _CHUNK)], sem))
    for d in cdescs:
      d.wait()

    def sample_body(s, c2):
      rbase = s * _C
      acc0 = zeros
      acc1 = zeros
      for c in range(_C):
        acc0 = acc0 + ctx_rows[rbase + c, pl.ds(0, 16)]
        acc1 = acc1 + ctx_rows[rbase + c, pl.ds(16, 16)]
      g = sc * _SUB + s
      mean_v[g, pl.ds(0, 16)] = acc0 * inv_c
      mean_v[g, pl.ds(16, 16)] = acc1 * inv_c
      return c2

    lax.fori_loop(0, _SUB, sample_body, 0)
    return carry

  lax.fori_loop(0, _NSUB, ctx_chunk, 0)

  def node_chunk(sc, carry):
    ndescs = []
    for j in range(_SUB * _L // _CHUNK):
      r0 = sc * (_SUB * _L // _CHUNK) + j
      ndescs.append(pltpu.async_copy(
          nodemb_hbm.at[flat_idx.at[r0]],
          node_rows.at[pl.ds(j * _CHUNK, _CHUNK)], sem))
    for d in ndescs:
      d.wait()

    for blk in range(_SUB // 16):
      s0 = sc * _SUB + blk * 16
      lanes = s0 + iota
      lens_t = plsc.load_gather(lens_v, [lanes // _CHUNK, lanes % _CHUNK])
      cb_t = plsc.load_gather(cb_v, [lanes // _CHUNK, lanes % _CHUNK])
      mean_t = [
          plsc.load_gather(mean_v, [lanes, jnp.full((16,), d_, jnp.int32)])
          for d_ in range(_D)
      ]
      row0 = (blk * 16 + iota) * _L

      def l_body(l, c2, row0=row0, lanes=lanes, lens_t=lens_t, cb_t=cb_t,
                 mean_t=mean_t):
        lv = jnp.full((16,), l, jnp.int32)
        acc = zeros
        for d_ in range(_D):
          nv = plsc.load_gather(node_rows,
                                [row0 + l, jnp.full((16,), d_, jnp.int32)])
          acc = acc + mean_t[d_] * nv
        code = lax.shift_right_logical(cb_t, l) & 1
        sign = code.astype(jnp.float32) * 2.0 - 1.0
        val = jnp.where(lv < lens_t, sign * acc, _FILL)
        plsc.store_scatter(scores_v, [lanes, lv], val)
        return c2

      lax.fori_loop(0, _L, l_body, 0)
    return carry

  lax.fori_loop(0, _NSUB, node_chunk, 0)

  pltpu.sync_copy(scores_v, out_hbm.at[pl.ds(base, _BW)])


_sc_scores = functools.partial(
    pl.kernel,
    out_type=jax.ShapeDtypeStruct((_B, _L), jnp.float32),
    mesh=plsc.VectorSubcoreMesh(core_axis_name="c", subcore_axis_name="s"),
    compiler_params=pltpu.CompilerParams(use_tc_tiling_on_sc=False,
                                         needs_layout_passes=False),
    scratch_types=[
        pltpu.VMEM((_BW // _CHUNK, _CHUNK), jnp.int32),
        pltpu.VMEM((_BW, _L), jnp.int32),
        pltpu.VMEM((_BW // _CHUNK, _CHUNK), jnp.int32),
        pltpu.VMEM((_BW // _CHUNK, _CHUNK), jnp.int32),
        pltpu.VMEM((_BW * _L // _CHUNK, _CHUNK), jnp.int32),
        pltpu.VMEM((_BW * _C // _CHUNK, _CHUNK), jnp.int32),
        pltpu.VMEM((_BW, _D), jnp.float32),
        pltpu.VMEM((_SUB * _C, _D), jnp.float32),
        pltpu.VMEM((_SUB * _L, _D), jnp.float32),
        pltpu.VMEM((_BW, _L), jnp.float32),
        pltpu.SemaphoreType.DMA,
    ],
)(_sc_body)


def _loss_body(x_ref, o_ref):
  x = x_ref[...]
  loss = jnp.log(1.0 + jnp.exp(-jnp.abs(x))) - jnp.minimum(x, 0.0)
  o_ref[0, 0] = jnp.sum(loss) * jnp.float32(1.0 / _B)


_loss = pl.pallas_call(
    _loss_body,
    out_shape=jax.ShapeDtypeStruct((1, 1), jnp.float32),
    out_specs=pl.BlockSpec(memory_space=pltpu.SMEM),
)


@jax.jit
def _impl(context_words, target_words, input_emb, internal_emb, paths, codes,
          path_lens):
  ctx_flat = context_words.astype(jnp.int32).reshape(_B * _C // _CHUNK, _CHUNK)
  tgt = target_words.astype(jnp.int32).reshape(_B // _CHUNK, _CHUNK)
  cbits = jnp.sum(codes.astype(jnp.int32) << jnp.arange(_L, dtype=jnp.int32)[None, :],
                  axis=1)
  scores = _sc_scores(ctx_flat, tgt, input_emb, internal_emb,
                      paths.astype(jnp.int32), cbits,
                      path_lens.astype(jnp.int32))
  loss = _loss(scores.reshape(_B * _L // _CHUNK, _CHUNK))
  return loss[0, 0]


def kernel(context_words, target_words, input_emb, internal_emb, paths, codes,
           path_lens):
  return _impl(context_words, target_words, input_emb, internal_emb, paths,
               codes, path_lens)
